# baseline (speedup 1.0000x reference)
import functools

import jax
import jax.numpy as jnp
from jax import lax
from jax.experimental import pallas as pl
from jax.experimental.pallas import tpu as pltpu
from jax.experimental.pallas import tpu_sc as plsc

_BETA = 0.25
_BN = 256
_CH = 128


def _dist_argmin_body(z_ref, cb_ref, idx_ref, loss_ref, en_ref, iota_ref):
    step = pl.program_id(0)
    nsteps = pl.num_programs(0)

    @pl.when(step == 0)
    def _():
        cb = cb_ref[...]
        en_ref[...] = jnp.sum(cb * cb, axis=1)[None, :]
        iota_ref[...] = lax.broadcasted_iota(
            jnp.int32, iota_ref.shape, 1
        ).astype(jnp.float32)

    z = z_ref[...]
    zn = jnp.sum(z * z, axis=1, keepdims=True)
    m2 = lax.dot_general(
        z * -2.0,
        cb_ref[...],
        (((1,), (1,)), ((), ())),
        preferred_element_type=jnp.float32,
    )
    d = (zn + en_ref[...]) + m2
    mind = jnp.min(d, axis=1, keepdims=True)
    k = d.shape[1]
    idxf = jnp.min(jnp.where(d <= mind, iota_ref[...], float(k)), axis=1)
    idx_ref[...] = idxf.astype(jnp.int32)

    bn = z.shape[0]
    dim = z.shape[1]
    scale = (1.0 + _BETA) / (bn * nsteps * dim)
    prev = jnp.where(step == 0, 0.0, loss_ref[0, 0])
    total = prev + jnp.sum(mind)
    loss_ref[0, 0] = jnp.where(step == nsteps - 1, total * scale, total)


def _make_sc_gather(n, k, d):
    info = plsc.get_sparse_core_info()
    nw = info.num_cores * info.num_subcores
    b_per_w = n // nw
    n_chunks = b_per_w // _CH
    mesh = plsc.VectorSubcoreMesh(core_axis_name="c", subcore_axis_name="s")

    @functools.partial(
        pl.kernel,
        mesh=mesh,
        out_type=jax.ShapeDtypeStruct((n, d), jnp.float32),
        scratch_types=[
            pltpu.VMEM((_CH,), jnp.int32),
            pltpu.VMEM((_CH, d), jnp.float32),
            pltpu.SemaphoreType.DMA,
        ],
        compiler_params=pltpu.CompilerParams(use_tc_tiling_on_sc=False),
    )
    def gather_kernel(table_hbm, idx_hbm, out_hbm, idx_v, rows_v, sem):
        wid = lax.axis_index("s") * info.num_cores + lax.axis_index("c")
        base = wid * b_per_w

        def body(i, carry):
            off = base + i * _CH
            pltpu.sync_copy(idx_hbm.at[pl.ds(off, _CH)], idx_v)
            pltpu.async_copy(table_hbm.at[idx_v], rows_v, sem).wait()
            pltpu.sync_copy(rows_v, out_hbm.at[pl.ds(off, _CH)])
            return carry

        lax.fori_loop(0, n_chunks, body, 0)

    return gather_kernel


@jax.jit
def kernel(z, codebook):
    n, d = z.shape
    k = codebook.shape[0]

    idx, loss = pl.pallas_call(
        _dist_argmin_body,
        grid=(n // _BN,),
        in_specs=[
            pl.BlockSpec((_BN, d), lambda i: (i, 0)),
            pl.BlockSpec((k, d), lambda i: (0, 0)),
        ],
        out_specs=[
            pl.BlockSpec((_BN,), lambda i: (i,)),
            pl.BlockSpec(memory_space=pltpu.SMEM),
        ],
        out_shape=[
            jax.ShapeDtypeStruct((n,), jnp.int32),
            jax.ShapeDtypeStruct((1, 1), jnp.float32),
        ],
        scratch_shapes=[
            pltpu.VMEM((1, k), jnp.float32),
            pltpu.VMEM((_BN, k), jnp.float32),
        ],
    )(z, codebook)

    zq = _make_sc_gather(n, k, d)(codebook, idx)
    return zq, idx, loss[0, 0]


# speedup vs baseline: 1.1840x; 1.1840x over previous
import functools

import jax
import jax.numpy as jnp
from jax import lax
from jax.experimental import pallas as pl
from jax.experimental.pallas import tpu as pltpu
from jax.experimental.pallas import tpu_sc as plsc

_BETA = 0.25
_BN = 256
_CH = 128


def _dist_argmin_body(z_ref, cbt_ref, idx_ref, loss_ref, en_ref):
    step = pl.program_id(0)
    nsteps = pl.num_programs(0)

    @pl.when(step == 0)
    def _():
        cbt = cbt_ref[...]
        en_ref[...] = jnp.sum(cbt * cbt, axis=0, keepdims=True)

    z = z_ref[...]
    zn = jnp.sum(z * z, axis=1, keepdims=True)
    m2 = jnp.dot(z * -2.0, cbt_ref[...], preferred_element_type=jnp.float32)
    d = (zn + en_ref[...]) + m2
    idx_ref[...] = jnp.argmin(d, axis=1).astype(jnp.int32)
    mind = jnp.min(d, axis=1)

    bn = z.shape[0]
    dim = z.shape[1]
    scale = (1.0 + _BETA) / (bn * nsteps * dim)
    prev = jnp.where(step == 0, 0.0, loss_ref[0, 0])
    total = prev + jnp.sum(mind)
    loss_ref[0, 0] = jnp.where(step == nsteps - 1, total * scale, total)


def _make_sc_gather(n, k, d):
    info = plsc.get_sparse_core_info()
    nw = info.num_cores * info.num_subcores
    b_per_w = n // nw
    n_chunks = b_per_w // _CH
    mesh = plsc.VectorSubcoreMesh(core_axis_name="c", subcore_axis_name="s")

    @functools.partial(
        pl.kernel,
        mesh=mesh,
        out_type=jax.ShapeDtypeStruct((n, d), jnp.float32),
        scratch_types=[
            pltpu.VMEM((2, _CH), jnp.int32),
            pltpu.VMEM((2, _CH, d), jnp.float32),
            pltpu.SemaphoreType.DMA,
            pltpu.SemaphoreType.DMA,
            pltpu.SemaphoreType.DMA,
            pltpu.SemaphoreType.DMA,
            pltpu.SemaphoreType.DMA,
            pltpu.SemaphoreType.DMA,
        ],
        compiler_params=pltpu.CompilerParams(use_tc_tiling_on_sc=False),
    )
    def gather_kernel(table_hbm, idx_hbm, out_hbm, idx_v, rows_v,
                      si0, si1, sg0, sg1, so0, so1):
        wid = lax.axis_index("s") * info.num_cores + lax.axis_index("c")
        base = wid * b_per_w
        si, sg, so = (si0, si1), (sg0, sg1), (so0, so1)

        def idx_start(i):
            b = i % 2
            return pltpu.async_copy(
                idx_hbm.at[pl.ds(base + i * _CH, _CH)], idx_v.at[b], si[b])

        hi = {0: idx_start(0)}
        hg, ho = {}, {}
        for i in range(n_chunks):
            b = i % 2
            if i + 1 < n_chunks:
                hi[i + 1] = idx_start(i + 1)
            hi[i].wait()
            if i >= 2:
                ho[i - 2].wait()
            hg[i] = pltpu.async_copy(
                table_hbm.at[idx_v.at[b]], rows_v.at[b], sg[b])
            hg[i].wait()
            ho[i] = pltpu.async_copy(
                rows_v.at[b], out_hbm.at[pl.ds(base + i * _CH, _CH)], so[b])
        for i in range(max(0, n_chunks - 2), n_chunks):
            ho[i].wait()

    return gather_kernel


@jax.jit
def kernel(z, codebook):
    n, d = z.shape
    k = codebook.shape[0]
    cbt = codebook.T

    idx, loss = pl.pallas_call(
        _dist_argmin_body,
        grid=(n // _BN,),
        in_specs=[
            pl.BlockSpec((_BN, d), lambda i: (i, 0)),
            pl.BlockSpec((d, k), lambda i: (0, 0)),
        ],
        out_specs=[
            pl.BlockSpec((_BN,), lambda i: (i,)),
            pl.BlockSpec(memory_space=pltpu.SMEM),
        ],
        out_shape=[
            jax.ShapeDtypeStruct((n,), jnp.int32),
            jax.ShapeDtypeStruct((1, 1), jnp.float32),
        ],
        scratch_shapes=[pltpu.VMEM((1, k), jnp.float32)],
    )(z, cbt)

    zq = _make_sc_gather(n, k, d)(codebook, idx)
    return zq, idx, loss[0, 0]


# speedup vs baseline: 1.3939x; 1.1773x over previous
import functools

import jax
import jax.numpy as jnp
from jax import lax
from jax.experimental import pallas as pl
from jax.experimental.pallas import tpu as pltpu
from jax.experimental.pallas import tpu_sc as plsc

_BETA = 0.25
_BN = 256
_CH = 128


def _dist_argmin_body(z_ref, cbt_ref, idx_ref, en_ref):
    step = pl.program_id(0)

    @pl.when(step == 0)
    def _():
        cbt = cbt_ref[...]
        en_ref[...] = jnp.sum(cbt * cbt, axis=0, keepdims=True)

    z = z_ref[...]
    zn = jnp.sum(z * z, axis=1, keepdims=True)
    m2 = jnp.dot(z * -2.0, cbt_ref[...], preferred_element_type=jnp.float32)
    d = (zn + en_ref[...]) + m2
    idx_ref[...] = jnp.argmin(d, axis=1).astype(jnp.int32)


def _loss_body(z_ref, zq_ref, loss_ref):
    step = pl.program_id(0)
    nsteps = pl.num_programs(0)
    e = z_ref[...] - zq_ref[...]
    s = jnp.sum(e * e)
    bn, dim = z_ref.shape
    scale = (1.0 + _BETA) / (bn * nsteps * dim)
    prev = jnp.where(step == 0, 0.0, loss_ref[0, 0])
    total = prev + s
    loss_ref[0, 0] = jnp.where(step == nsteps - 1, total * scale, total)


def _make_sc_gather(n, k, d):
    info = plsc.get_sparse_core_info()
    nw = info.num_cores * info.num_subcores
    b_per_w = n // nw
    n_chunks = b_per_w // _CH
    mesh = plsc.VectorSubcoreMesh(core_axis_name="c", subcore_axis_name="s")

    @functools.partial(
        pl.kernel,
        mesh=mesh,
        out_type=jax.ShapeDtypeStruct((n, d), jnp.float32),
        scratch_types=[
            pltpu.VMEM((2, _CH), jnp.int32),
            pltpu.VMEM((2, _CH, d), jnp.float32),
            pltpu.SemaphoreType.DMA,
            pltpu.SemaphoreType.DMA,
            pltpu.SemaphoreType.DMA,
            pltpu.SemaphoreType.DMA,
            pltpu.SemaphoreType.DMA,
            pltpu.SemaphoreType.DMA,
        ],
        compiler_params=pltpu.CompilerParams(use_tc_tiling_on_sc=False),
    )
    def gather_kernel(table_hbm, idx_hbm, out_hbm, idx_v, rows_v,
                      si0, si1, sg0, sg1, so0, so1):
        wid = lax.axis_index("s") * info.num_cores + lax.axis_index("c")
        base = wid * b_per_w
        si, sg, so = (si0, si1), (sg0, sg1), (so0, so1)

        def idx_start(i):
            b = i % 2
            return pltpu.async_copy(
                idx_hbm.at[pl.ds(base + i * _CH, _CH)], idx_v.at[b], si[b])

        hi = {0: idx_start(0)}
        hg, ho = {}, {}
        for i in range(n_chunks):
            b = i % 2
            if i + 1 < n_chunks:
                hi[i + 1] = idx_start(i + 1)
            hi[i].wait()
            if i >= 2:
                ho[i - 2].wait()
            hg[i] = pltpu.async_copy(
                table_hbm.at[idx_v.at[b]], rows_v.at[b], sg[b])
            hg[i].wait()
            ho[i] = pltpu.async_copy(
                rows_v.at[b], out_hbm.at[pl.ds(base + i * _CH, _CH)], so[b])
        for i in range(max(0, n_chunks - 2), n_chunks):
            ho[i].wait()

    return gather_kernel


@jax.jit
def kernel(z, codebook):
    n, d = z.shape
    k = codebook.shape[0]
    cbt = codebook.T

    idx = pl.pallas_call(
        _dist_argmin_body,
        grid=(n // _BN,),
        in_specs=[
            pl.BlockSpec((_BN, d), lambda i: (i, 0)),
            pl.BlockSpec((d, k), lambda i: (0, 0)),
        ],
        out_specs=pl.BlockSpec((_BN,), lambda i: (i,)),
        out_shape=jax.ShapeDtypeStruct((n,), jnp.int32),
        scratch_shapes=[pltpu.VMEM((1, k), jnp.float32)],
    )(z, cbt)

    zq = _make_sc_gather(n, k, d)(codebook, idx)

    bl = 4096
    loss = pl.pallas_call(
        _loss_body,
        grid=(n // bl,),
        in_specs=[
            pl.BlockSpec((bl, d), lambda i: (i, 0)),
            pl.BlockSpec((bl, d), lambda i: (i, 0)),
        ],
        out_specs=pl.BlockSpec(memory_space=pltpu.SMEM),
        out_shape=jax.ShapeDtypeStruct((1, 1), jnp.float32),
    )(z, zq)
    return zq, idx, loss[0, 0]
